# baseline (speedup 1.0000x reference)
import functools

import jax
import jax.numpy as jnp
from jax import lax
from jax.experimental import pallas as pl
from jax.experimental.pallas import tpu as pltpu
from jax.experimental.pallas import tpu_sc as plsc

_CHUNK = 112
_NBUF = 8


def _make_gather(n_sites, d_feat, n_points):
    info = plsc.get_sparse_core_info()
    nc, ns = info.num_cores, info.num_subcores
    nw = nc * ns

    chunk = _CHUNK
    nbuf = _NBUF
    per_w = -(-n_points // nw)
    per_w = -(-per_w // chunk) * chunk
    k = per_w // chunk

    mesh = plsc.VectorSubcoreMesh(core_axis_name="c", subcore_axis_name="s")

    scratch = [pltpu.VMEM((per_w,), jnp.int32)]
    scratch += [pltpu.VMEM((chunk, d_feat), jnp.float32)] * nbuf
    scratch += [pltpu.SemaphoreType.DMA] * (2 * nbuf)

    @functools.partial(
        pl.kernel,
        mesh=mesh,
        out_type=jax.ShapeDtypeStruct((n_points, d_feat), jnp.float32),
        scratch_types=scratch,
    )
    def gather_kernel(table_hbm, idx_hbm, out_hbm, idx_v, *bufs_sems):
        rows_v = bufs_sems[:nbuf]
        gsem = bufs_sems[nbuf:2 * nbuf]
        ssem = bufs_sems[2 * nbuf:]
        wid = lax.axis_index("s") * nc + lax.axis_index("c")
        base = jnp.minimum(wid * per_w, n_points - per_w)

        pltpu.sync_copy(idx_hbm.at[pl.ds(base, per_w)], idx_v)

        look = nbuf // 2
        gathers = [None] * nbuf
        stores = [None] * nbuf
        for j in range(min(look, k)):
            gathers[j] = pltpu.async_copy(
                table_hbm.at[idx_v.at[pl.ds(j * chunk, chunk)]],
                rows_v[j], gsem[j])
        for j in range(k):
            p = j % nbuf
            jn = j + look
            if jn < k:
                q = jn % nbuf
                if stores[q] is not None:
                    stores[q].wait()
                gathers[q] = pltpu.async_copy(
                    table_hbm.at[idx_v.at[pl.ds(jn * chunk, chunk)]],
                    rows_v[q], gsem[q])
            gathers[p].wait()
            stores[p] = pltpu.async_copy(
                rows_v[p], out_hbm.at[pl.ds(base + j * chunk, chunk)],
                ssem[p])
        for j in range(max(0, k - nbuf), k):
            if stores[j % nbuf] is not None:
                stores[j % nbuf].wait()

    return gather_kernel


def kernel(features, point_to_site):
    n_sites, d_feat = features.shape
    (n_points,) = point_to_site.shape
    return _make_gather(n_sites, d_feat, n_points)(features, point_to_site)


# speedup vs baseline: 1.0148x; 1.0148x over previous
import functools

import jax
import jax.numpy as jnp
from jax import lax
from jax.experimental import pallas as pl
from jax.experimental.pallas import tpu as pltpu
from jax.experimental.pallas import tpu_sc as plsc

_CHUNK = 224
_NBUF = 4


def _make_gather(n_sites, d_feat, n_points):
    info = plsc.get_sparse_core_info()
    nc, ns = info.num_cores, info.num_subcores
    nw = nc * ns

    chunk = _CHUNK
    nbuf = _NBUF
    per_w = -(-n_points // nw)
    per_w = -(-per_w // chunk) * chunk
    k = per_w // chunk

    mesh = plsc.VectorSubcoreMesh(core_axis_name="c", subcore_axis_name="s")

    scratch = [pltpu.VMEM((per_w,), jnp.int32)]
    scratch += [pltpu.VMEM((chunk, d_feat), jnp.float32)] * nbuf
    scratch += [pltpu.SemaphoreType.DMA] * (2 * nbuf)

    @functools.partial(
        pl.kernel,
        mesh=mesh,
        out_type=jax.ShapeDtypeStruct((n_points, d_feat), jnp.float32),
        scratch_types=scratch,
    )
    def gather_kernel(table_hbm, idx_hbm, out_hbm, idx_v, *bufs_sems):
        rows_v = bufs_sems[:nbuf]
        gsem = bufs_sems[nbuf:2 * nbuf]
        ssem = bufs_sems[2 * nbuf:]
        wid = lax.axis_index("s") * nc + lax.axis_index("c")
        base = jnp.minimum(wid * per_w, n_points - per_w)

        pltpu.sync_copy(idx_hbm.at[pl.ds(base, per_w)], idx_v)

        look = nbuf // 2
        gathers = [None] * nbuf
        stores = [None] * nbuf
        for j in range(min(look, k)):
            gathers[j] = pltpu.async_copy(
                table_hbm.at[idx_v.at[pl.ds(j * chunk, chunk)]],
                rows_v[j], gsem[j])
        for j in range(k):
            p = j % nbuf
            jn = j + look
            if jn < k:
                q = jn % nbuf
                if stores[q] is not None:
                    stores[q].wait()
                gathers[q] = pltpu.async_copy(
                    table_hbm.at[idx_v.at[pl.ds(jn * chunk, chunk)]],
                    rows_v[q], gsem[q])
            gathers[p].wait()
            stores[p] = pltpu.async_copy(
                rows_v[p], out_hbm.at[pl.ds(base + j * chunk, chunk)],
                ssem[p])
        for j in range(max(0, k - nbuf), k):
            if stores[j % nbuf] is not None:
                stores[j % nbuf].wait()

    return gather_kernel


def kernel(features, point_to_site):
    n_sites, d_feat = features.shape
    (n_points,) = point_to_site.shape
    return _make_gather(n_sites, d_feat, n_points)(features, point_to_site)
